# baseline (speedup 1.0000x reference)
import functools

import jax
import jax.numpy as jnp
from jax import lax
from jax.experimental import pallas as pl
from jax.experimental.pallas import tpu as pltpu
from jax.experimental.pallas import tpu_sc as plsc

_D = 128
_DW = _D // 2
_NC = 2
_NS = 16
_NW = _NC * _NS
_CH = 256


def _tc_transform_body(dk_ref, x_ref, g_ref, o_ref):
    x = x_ref[...] * dk_ref[...]
    z = jnp.dot(x, g_ref[...], preferred_element_type=jnp.float32)
    o_ref[...] = z * dk_ref[...]


def _transform_rows(x, g, dk):
    n = x.shape[0]
    blk = 1000
    assert n % blk == 0
    return pl.pallas_call(
        _tc_transform_body,
        grid=(n // blk,),
        in_specs=[
            pl.BlockSpec((1, _D), lambda i: (0, 0)),
            pl.BlockSpec((blk, _D), lambda i: (i, 0)),
            pl.BlockSpec((_D, _D), lambda i: (0, 0)),
        ],
        out_specs=pl.BlockSpec((blk, _D), lambda i: (i, 0)),
        out_shape=jax.ShapeDtypeStruct((n, _D), jnp.float32),
    )(dk.reshape(1, _D), x, g)


def _pack_bf16(x):
    n = x.shape[0]
    xb = x.astype(jnp.bfloat16).reshape(n, _DW, 2)
    return lax.bitcast_convert_type(xb, jnp.int32)


def _sc_decode(zp, cp, er, ec):
    e_total = er.shape[0]
    assert e_total % (_NW * _CH * 2) == 0
    epw = e_total // _NW
    nchunks = epw // _CH
    mesh = plsc.VectorSubcoreMesh(core_axis_name="c", subcore_axis_name="s")

    @functools.partial(
        pl.kernel,
        mesh=mesh,
        compiler_params=pltpu.CompilerParams(
            needs_layout_passes=False, use_tc_tiling_on_sc=False),
        out_type=jax.ShapeDtypeStruct((e_total,), jnp.float32),
        scratch_types=[
            pltpu.VMEM((epw,), jnp.int32),
            pltpu.VMEM((epw,), jnp.int32),
            pltpu.VMEM((_CH, _DW), jnp.int32),
            pltpu.VMEM((_CH, _DW), jnp.int32),
            pltpu.VMEM((_CH, _DW), jnp.int32),
            pltpu.VMEM((_CH, _DW), jnp.int32),
            pltpu.VMEM((epw,), jnp.float32),
            pltpu.SemaphoreType.DMA,
            pltpu.SemaphoreType.DMA,
            pltpu.SemaphoreType.DMA,
            pltpu.SemaphoreType.DMA,
        ],
    )
    def k(z_hbm, c_hbm, er_hbm, ec_hbm, out_hbm,
          ir_v, ic_v, r0, r1, c0, c1, o_v, sr0, sr1, sc0, sc1):
        rbuf = (r0, r1)
        cbuf = (c0, c1)
        srs = (sr0, sr1)
        scs = (sc0, sc1)
        wid = lax.axis_index("s") * _NC + lax.axis_index("c")
        base = pl.multiple_of(wid * epw, 8)
        pltpu.sync_copy(er_hbm.at[pl.ds(base, epw)], ir_v)
        pltpu.sync_copy(ec_hbm.at[pl.ds(base, epw)], ic_v)

        lanes = lax.iota(jnp.int32, 16)

        def copies(ci, slot):
            off = pl.multiple_of(ci * _CH, 8)
            cr = pltpu.make_async_copy(
                z_hbm.at[ir_v.at[pl.ds(off, _CH)]], rbuf[slot], srs[slot])
            cc = pltpu.make_async_copy(
                c_hbm.at[ic_v.at[pl.ds(off, _CH)]], cbuf[slot], scs[slot])
            return cr, cc

        def start(ci, slot):
            cr, cc = copies(ci, slot)
            cr.start()
            cc.start()

        def wait(ci, slot):
            cr, cc = copies(ci, slot)
            cr.wait()
            cc.wait()

        def compute(ci, slot):
            rb = rbuf[slot]
            cb = cbuf[slot]
            off = pl.multiple_of(ci * _CH, 8)

            def ebody(t, res):
                e0 = t * 4
                g16 = (t // 4) * 16
                for u in range(4):
                    acc0 = None
                    acc1 = None
                    for kk in range(_DW // 16):
                        rw = plsc.bitcast(rb[e0 + u, pl.ds(kk * 16, 16)],
                                          jnp.bfloat16)
                        cw = plsc.bitcast(cb[e0 + u, pl.ds(kk * 16, 16)],
                                          jnp.bfloat16)
                        ra, rb2 = plsc.unpack(
                            rw, format=plsc.PackFormat.INTERLEAVED)
                        ca, cb2 = plsc.unpack(
                            cw, format=plsc.PackFormat.INTERLEAVED)
                        pa = ra * ca
                        pb = rb2 * cb2
                        acc0 = pa if acc0 is None else acc0 + pa
                        acc1 = pb if acc1 is None else acc1 + pb
                    s = jnp.sum(acc0 + acc1)
                    res = jnp.where(lanes == (t % 4) * 4 + u, s, res)
                o_v[pl.ds(off + g16, 16)] = res
                return res

            lax.fori_loop(0, _CH // 4, ebody,
                          jnp.zeros((16,), jnp.float32))

        start(0, 0)

        def body(p, carry):
            ci = p * 2
            start(ci + 1, 1)
            wait(ci, 0)
            compute(ci, 0)
            start(ci + 2, 0)
            wait(ci + 1, 1)
            compute(ci + 1, 1)
            return carry

        lax.fori_loop(0, (nchunks - 2) // 2, body, 0)
        start(nchunks - 1, 1)
        wait(nchunks - 2, 0)
        compute(nchunks - 2, 0)
        wait(nchunks - 1, 1)
        compute(nchunks - 1, 1)
        pltpu.sync_copy(o_v, out_hbm.at[pl.ds(base, epw)])

    return k(zp, cp, er, ec)


def kernel(rt_k, edges_row, edges_col, embeds_row, embeds_col, global_mat,
           local_diag):
    dk = lax.dynamic_index_in_dim(local_diag, rt_k, axis=0, keepdims=False)
    z = _transform_rows(embeds_row, global_mat, dk)
    zp = _pack_bf16(z)
    cp = _pack_bf16(embeds_col)
    e = edges_row.shape[0]
    quantum = _NW * _CH * 2
    e_pad = ((e + quantum - 1) // quantum) * quantum
    er = edges_row.astype(jnp.int32)
    ec = edges_col.astype(jnp.int32)
    if e_pad != e:
        pad = jnp.zeros((e_pad - e,), jnp.int32)
        er = jnp.concatenate([er, pad])
        ec = jnp.concatenate([ec, pad])
    preds = _sc_decode(zp, cp, er, ec)
    return preds[:e] if e_pad != e else preds


# speedup vs baseline: 1.5123x; 1.5123x over previous
import functools

import jax
import jax.numpy as jnp
from jax import lax
from jax.experimental import pallas as pl
from jax.experimental.pallas import tpu as pltpu
from jax.experimental.pallas import tpu_sc as plsc

_D = 128
_DW = _D // 2
_NC = 2
_NS = 16
_NW = _NC * _NS
_CH = 80


def _tc_transform_body(dk_ref, x_ref, g_ref, o_ref):
    x = x_ref[...] * dk_ref[...]
    z = jnp.dot(x, g_ref[...], preferred_element_type=jnp.float32)
    o_ref[...] = z * dk_ref[...]


def _transform_rows(x, g, dk):
    n = x.shape[0]
    blk = 1000
    assert n % blk == 0
    return pl.pallas_call(
        _tc_transform_body,
        grid=(n // blk,),
        in_specs=[
            pl.BlockSpec((1, _D), lambda i: (0, 0)),
            pl.BlockSpec((blk, _D), lambda i: (i, 0)),
            pl.BlockSpec((_D, _D), lambda i: (0, 0)),
        ],
        out_specs=pl.BlockSpec((blk, _D), lambda i: (i, 0)),
        out_shape=jax.ShapeDtypeStruct((n, _D), jnp.float32),
    )(dk.reshape(1, _D), x, g)


def _pack_bf16(x):
    n = x.shape[0]
    xb = x.astype(jnp.bfloat16).reshape(n, _DW, 2)
    xi = lax.bitcast_convert_type(xb, jnp.int32)
    return jnp.concatenate([xi, jnp.zeros((n, _D - _DW), jnp.int32)], axis=1)


def _sc_decode(zp, cp, er, ec):
    e_total = er.shape[0]
    assert e_total % (_NW * _CH) == 0
    epw = e_total // _NW
    nchunks = epw // _CH
    assert nchunks % 2 == 1
    mesh = plsc.VectorSubcoreMesh(core_axis_name="c", subcore_axis_name="s")

    @functools.partial(
        pl.kernel,
        mesh=mesh,
        compiler_params=pltpu.CompilerParams(needs_layout_passes=False),
        out_type=jax.ShapeDtypeStruct((e_total,), jnp.float32),
        scratch_types=[
            pltpu.VMEM((epw,), jnp.int32),
            pltpu.VMEM((epw,), jnp.int32),
            pltpu.VMEM((_CH, _D), jnp.int32),
            pltpu.VMEM((_CH, _D), jnp.int32),
            pltpu.VMEM((_CH, _D), jnp.int32),
            pltpu.VMEM((_CH, _D), jnp.int32),
            pltpu.VMEM((epw,), jnp.float32),
            pltpu.SemaphoreType.DMA,
            pltpu.SemaphoreType.DMA,
            pltpu.SemaphoreType.DMA,
            pltpu.SemaphoreType.DMA,
        ],
    )
    def k(z_hbm, c_hbm, er_hbm, ec_hbm, out_hbm,
          ir_v, ic_v, r0, r1, c0, c1, o_v, sr0, sr1, sc0, sc1):
        rbuf = (r0, r1)
        cbuf = (c0, c1)
        srs = (sr0, sr1)
        scs = (sc0, sc1)
        wid = lax.axis_index("s") * _NC + lax.axis_index("c")
        base = pl.multiple_of(wid * epw, 8)
        pltpu.sync_copy(er_hbm.at[pl.ds(base, epw)], ir_v)
        pltpu.sync_copy(ec_hbm.at[pl.ds(base, epw)], ic_v)

        lanes = lax.iota(jnp.int32, 16)

        def copies(ci, slot):
            off = pl.multiple_of(ci * _CH, 8)
            cr = pltpu.make_async_copy(
                z_hbm.at[ir_v.at[pl.ds(off, _CH)]], rbuf[slot], srs[slot])
            cc = pltpu.make_async_copy(
                c_hbm.at[ic_v.at[pl.ds(off, _CH)]], cbuf[slot], scs[slot])
            return cr, cc

        def start(ci, slot):
            cr, cc = copies(ci, slot)
            cr.start()
            cc.start()

        def wait(ci, slot):
            cr, cc = copies(ci, slot)
            cr.wait()
            cc.wait()

        def compute(ci, slot):
            rb = rbuf[slot]
            cb = cbuf[slot]
            off = pl.multiple_of(ci * _CH, 8)

            def ebody(t, res):
                e0 = t * 4
                g16 = (t // 4) * 16
                for u in range(4):
                    acc0 = None
                    acc1 = None
                    for kk in range(_DW // 16):
                        rw = plsc.bitcast(rb[e0 + u, pl.ds(kk * 16, 16)],
                                          jnp.bfloat16)
                        cw = plsc.bitcast(cb[e0 + u, pl.ds(kk * 16, 16)],
                                          jnp.bfloat16)
                        ra, rb2 = plsc.unpack(
                            rw, format=plsc.PackFormat.INTERLEAVED)
                        ca, cb2 = plsc.unpack(
                            cw, format=plsc.PackFormat.INTERLEAVED)
                        pa = ra * ca
                        pb = rb2 * cb2
                        acc0 = pa if acc0 is None else acc0 + pa
                        acc1 = pb if acc1 is None else acc1 + pb
                    s = jnp.sum(acc0 + acc1)
                    res = jnp.where(lanes == (t % 4) * 4 + u, s, res)
                o_v[pl.ds(off + g16, 16)] = res
                return res

            lax.fori_loop(0, _CH // 4, ebody,
                          jnp.zeros((16,), jnp.float32))

        start(0, 0)

        def body(p, carry):
            ci = p * 2
            start(ci + 1, 1)
            wait(ci, 0)
            compute(ci, 0)
            start(ci + 2, 0)
            wait(ci + 1, 1)
            compute(ci + 1, 1)
            return carry

        lax.fori_loop(0, (nchunks - 1) // 2, body, 0)
        wait(nchunks - 1, 0)
        compute(nchunks - 1, 0)
        pltpu.sync_copy(o_v, out_hbm.at[pl.ds(base, epw)])

    return k(zp, cp, er, ec)


def kernel(rt_k, edges_row, edges_col, embeds_row, embeds_col, global_mat,
           local_diag):
    dk = lax.dynamic_index_in_dim(local_diag, rt_k, axis=0, keepdims=False)
    z = _transform_rows(embeds_row, global_mat, dk)
    zp = _pack_bf16(z)
    cp = _pack_bf16(embeds_col)
    er = edges_row.astype(jnp.int32)
    ec = edges_col.astype(jnp.int32)
    return _sc_decode(zp, cp, er, ec)


# speedup vs baseline: 1.9471x; 1.2875x over previous
import functools

import jax
import jax.numpy as jnp
from jax import lax
from jax.experimental import pallas as pl
from jax.experimental.pallas import tpu as pltpu
from jax.experimental.pallas import tpu_sc as plsc

_D = 128
_DW = _D // 2
_NC = 2
_NS = 16
_NW = _NC * _NS
_CH = 80


def _tc_transform_body(dk_ref, x_ref, g_ref, o_ref):
    x = x_ref[...] * dk_ref[...]
    z = jnp.dot(x, g_ref[...], preferred_element_type=jnp.float32)
    o_ref[...] = z * dk_ref[...]


def _transform_rows(x, g, dk):
    n = x.shape[0]
    blk = 1000
    assert n % blk == 0
    return pl.pallas_call(
        _tc_transform_body,
        grid=(n // blk,),
        in_specs=[
            pl.BlockSpec((1, _D), lambda i: (0, 0)),
            pl.BlockSpec((blk, _D), lambda i: (i, 0)),
            pl.BlockSpec((_D, _D), lambda i: (0, 0)),
        ],
        out_specs=pl.BlockSpec((blk, _D), lambda i: (i, 0)),
        out_shape=jax.ShapeDtypeStruct((n, _D), jnp.float32),
    )(dk.reshape(1, _D), x, g)


def _sc_decode(zp, cp, er, ec):
    e_total = er.shape[0]
    assert e_total % (_NW * _CH) == 0
    epw = e_total // _NW
    nchunks = epw // _CH
    assert nchunks % 2 == 1
    mesh = plsc.VectorSubcoreMesh(core_axis_name="c", subcore_axis_name="s")

    @functools.partial(
        pl.kernel,
        mesh=mesh,
        compiler_params=pltpu.CompilerParams(needs_layout_passes=False),
        out_type=jax.ShapeDtypeStruct((e_total,), jnp.float32),
        scratch_types=[
            pltpu.VMEM((epw,), jnp.int32),
            pltpu.VMEM((epw,), jnp.int32),
            pltpu.VMEM((_CH, _D), jnp.float32),
            pltpu.VMEM((_CH, _D), jnp.float32),
            pltpu.VMEM((_CH, _D), jnp.float32),
            pltpu.VMEM((_CH, _D), jnp.float32),
            pltpu.VMEM((epw,), jnp.float32),
            pltpu.SemaphoreType.DMA,
            pltpu.SemaphoreType.DMA,
            pltpu.SemaphoreType.DMA,
            pltpu.SemaphoreType.DMA,
        ],
    )
    def k(z_hbm, c_hbm, er_hbm, ec_hbm, out_hbm,
          ir_v, ic_v, r0, r1, c0, c1, o_v, sr0, sr1, sc0, sc1):
        rbuf = (r0, r1)
        cbuf = (c0, c1)
        srs = (sr0, sr1)
        scs = (sc0, sc1)
        wid = lax.axis_index("s") * _NC + lax.axis_index("c")
        base = pl.multiple_of(wid * epw, 8)
        pltpu.sync_copy(er_hbm.at[pl.ds(base, epw)], ir_v)
        pltpu.sync_copy(ec_hbm.at[pl.ds(base, epw)], ic_v)

        lanes = lax.iota(jnp.int32, 16)

        def copies(ci, slot):
            off = pl.multiple_of(ci * _CH, 8)
            cr = pltpu.make_async_copy(
                z_hbm.at[ir_v.at[pl.ds(off, _CH)]], rbuf[slot], srs[slot])
            cc = pltpu.make_async_copy(
                c_hbm.at[ic_v.at[pl.ds(off, _CH)]], cbuf[slot], scs[slot])
            return cr, cc

        def start(ci, slot):
            cr, cc = copies(ci, slot)
            cr.start()
            cc.start()

        def wait(ci, slot):
            cr, cc = copies(ci, slot)
            cr.wait()
            cc.wait()

        def compute(ci, slot):
            rb = rbuf[slot]
            cb = cbuf[slot]
            off = pl.multiple_of(ci * _CH, 8)

            def ebody(t, res):
                e0 = t * 4
                g16 = (t // 4) * 16
                for u in range(4):
                    acc0 = rb[e0 + u, pl.ds(0, 16)] * cb[e0 + u, pl.ds(0, 16)]
                    acc1 = rb[e0 + u, pl.ds(16, 16)] * cb[e0 + u, pl.ds(16, 16)]
                    for kk in range(2, _D // 16):
                        seg = rb[e0 + u, pl.ds(kk * 16, 16)]
                        seg = seg * cb[e0 + u, pl.ds(kk * 16, 16)]
                        if kk % 2 == 0:
                            acc0 = acc0 + seg
                        else:
                            acc1 = acc1 + seg
                    s = jnp.sum(acc0 + acc1)
                    res = jnp.where(lanes == (t % 4) * 4 + u, s, res)
                o_v[pl.ds(off + g16, 16)] = res
                return res

            lax.fori_loop(0, _CH // 4, ebody,
                          jnp.zeros((16,), jnp.float32))

        start(0, 0)

        def body(p, carry):
            ci = p * 2
            start(ci + 1, 1)
            wait(ci, 0)
            compute(ci, 0)
            start(ci + 2, 0)
            wait(ci + 1, 1)
            compute(ci + 1, 1)
            return carry

        lax.fori_loop(0, (nchunks - 1) // 2, body, 0)
        wait(nchunks - 1, 0)
        compute(nchunks - 1, 0)
        pltpu.sync_copy(o_v, out_hbm.at[pl.ds(base, epw)])

    return k(zp, cp, er, ec)


def kernel(rt_k, edges_row, edges_col, embeds_row, embeds_col, global_mat,
           local_diag):
    dk = lax.dynamic_index_in_dim(local_diag, rt_k, axis=0, keepdims=False)
    z = _transform_rows(embeds_row, global_mat, dk)
    er = edges_row.astype(jnp.int32)
    ec = edges_col.astype(jnp.int32)
    return _sc_decode(z, embeds_col, er, ec)


# speedup vs baseline: 2.0734x; 1.0649x over previous
import functools

import jax
import jax.numpy as jnp
from jax import lax
from jax.experimental import pallas as pl
from jax.experimental.pallas import tpu as pltpu
from jax.experimental.pallas import tpu_sc as plsc

_D = 128
_DW = _D // 2
_NC = 2
_NS = 16
_NW = _NC * _NS
_CH = 80


def _tc_transform_body(dk_ref, x_ref, g_ref, o_ref):
    x = x_ref[...] * dk_ref[...]
    z = jnp.dot(x, g_ref[...], preferred_element_type=jnp.float32)
    o_ref[...] = z * dk_ref[...]


def _transform_rows(x, g, dk):
    n = x.shape[0]
    blk = 1000
    assert n % blk == 0
    return pl.pallas_call(
        _tc_transform_body,
        grid=(n // blk,),
        in_specs=[
            pl.BlockSpec((1, _D), lambda i: (0, 0)),
            pl.BlockSpec((blk, _D), lambda i: (i, 0)),
            pl.BlockSpec((_D, _D), lambda i: (0, 0)),
        ],
        out_specs=pl.BlockSpec((blk, _D), lambda i: (i, 0)),
        out_shape=jax.ShapeDtypeStruct((n, _D), jnp.float32),
    )(dk.reshape(1, _D), x, g)


def _sc_decode(zp, cp, er, ec):
    e_total = er.shape[0]
    assert e_total % (_NW * _CH) == 0
    epw = e_total // _NW
    nchunks = epw // _CH
    assert nchunks % 2 == 1
    mesh = plsc.VectorSubcoreMesh(core_axis_name="c", subcore_axis_name="s")

    @functools.partial(
        pl.kernel,
        mesh=mesh,
        compiler_params=pltpu.CompilerParams(needs_layout_passes=False),
        out_type=jax.ShapeDtypeStruct((e_total,), jnp.float32),
        scratch_types=[
            pltpu.VMEM((epw,), jnp.int32),
            pltpu.VMEM((epw,), jnp.int32),
            pltpu.VMEM((_CH, _D), jnp.float32),
            pltpu.VMEM((_CH, _D), jnp.float32),
            pltpu.VMEM((_CH, _D), jnp.float32),
            pltpu.VMEM((_CH, _D), jnp.float32),
            pltpu.VMEM((epw,), jnp.float32),
            pltpu.SemaphoreType.DMA,
            pltpu.SemaphoreType.DMA,
            pltpu.SemaphoreType.DMA,
            pltpu.SemaphoreType.DMA,
        ],
    )
    def k(z_hbm, c_hbm, er_hbm, ec_hbm, out_hbm,
          ir_v, ic_v, r0, r1, c0, c1, o_v, sr0, sr1, sc0, sc1):
        rbuf = (r0, r1)
        cbuf = (c0, c1)
        srs = (sr0, sr1)
        scs = (sc0, sc1)
        wid = lax.axis_index("s") * _NC + lax.axis_index("c")
        base = pl.multiple_of(wid * epw, 8)
        pltpu.sync_copy(er_hbm.at[pl.ds(base, epw)], ir_v)
        pltpu.sync_copy(ec_hbm.at[pl.ds(base, epw)], ic_v)

        lanes = lax.iota(jnp.int32, 16)

        def copies(ci, slot):
            off = pl.multiple_of(ci * _CH, 8)
            cr = pltpu.make_async_copy(
                z_hbm.at[ir_v.at[pl.ds(off, _CH)]], rbuf[slot], srs[slot])
            cc = pltpu.make_async_copy(
                c_hbm.at[ic_v.at[pl.ds(off, _CH)]], cbuf[slot], scs[slot])
            return cr, cc

        def start(ci, slot):
            cr, cc = copies(ci, slot)
            cr.start()
            cc.start()

        def wait(ci, slot):
            cr, cc = copies(ci, slot)
            cr.wait()
            cc.wait()

        def compute(ci, slot):
            rb = rbuf[slot]
            cb = cbuf[slot]
            off = pl.multiple_of(ci * _CH, 8)

            mask = lanes < 4

            @plsc.parallel_loop(0, _CH // 4, unroll=2)
            def _(t):
                e0 = t * 4
                ss = []
                for u in range(4):
                    acc0 = rb[e0 + u, pl.ds(0, 16)] * cb[e0 + u, pl.ds(0, 16)]
                    acc1 = rb[e0 + u, pl.ds(16, 16)] * cb[e0 + u, pl.ds(16, 16)]
                    for kk in range(2, _D // 16):
                        seg = rb[e0 + u, pl.ds(kk * 16, 16)]
                        seg = seg * cb[e0 + u, pl.ds(kk * 16, 16)]
                        if kk % 2 == 0:
                            acc0 = acc0 + seg
                        else:
                            acc1 = acc1 + seg
                    ss.append(jnp.sum(acc0 + acc1))
                vals = jnp.where(
                    lanes == 0, ss[0],
                    jnp.where(lanes == 1, ss[1],
                              jnp.where(lanes == 2, ss[2], ss[3])))
                plsc.store_scatter(o_v, [off + e0 + lanes], vals, mask=mask)

        start(0, 0)

        def body(p, carry):
            ci = p * 2
            start(ci + 1, 1)
            wait(ci, 0)
            compute(ci, 0)
            start(ci + 2, 0)
            wait(ci + 1, 1)
            compute(ci + 1, 1)
            return carry

        lax.fori_loop(0, (nchunks - 1) // 2, body, 0)
        wait(nchunks - 1, 0)
        compute(nchunks - 1, 0)
        pltpu.sync_copy(o_v, out_hbm.at[pl.ds(base, epw)])

    return k(zp, cp, er, ec)


def kernel(rt_k, edges_row, edges_col, embeds_row, embeds_col, global_mat,
           local_diag):
    dk = lax.dynamic_index_in_dim(local_diag, rt_k, axis=0, keepdims=False)
    z = _transform_rows(embeds_row, global_mat, dk)
    er = edges_row.astype(jnp.int32)
    ec = edges_col.astype(jnp.int32)
    return _sc_decode(z, embeds_col, er, ec)
